# baseline (speedup 1.0000x reference)
import functools

import jax
import jax.numpy as jnp
from jax import lax
from jax.experimental import pallas as pl
from jax.experimental.pallas import tpu as pltpu
from jax.experimental.pallas import tpu_sc as plsc

B = 128
V = 100000

TC_CHUNK = 4096
TC_NCHUNK = 12
C0 = TC_CHUNK * TC_NCHUNK
W = V - C0

LANES = 128
FOLDS = TC_CHUNK // LANES
_NEG_INF = float("-inf")

SC_NC = 2
SC_NS = 16
NW = SC_NC * SC_NS
ROWS_PER = B // NW
SC_L = 16


def _tc_kernel(label_ref, logits_ref, outmax_ref, outlab_ref,
               accmax_ref, acclab_ref):
    i = pl.program_id(0)
    x = logits_ref[...]
    lane = jax.lax.broadcasted_iota(jnp.int32, (B, LANES), 1)
    lab_rel = label_ref[...].reshape(B, 1) - i * TC_CHUNK

    @pl.when(i == 0)
    def _init():
        accmax_ref[...] = jnp.full((B, LANES), _NEG_INF, jnp.float32)
        acclab_ref[...] = jnp.full((B, LANES), _NEG_INF, jnp.float32)

    acc_m = accmax_ref[...]
    acc_l = acclab_ref[...]
    for k in range(FOLDS):
        xs = x[:, k * LANES:(k + 1) * LANES]
        is_lab = lane == (lab_rel - k * LANES)
        acc_m = jnp.maximum(acc_m, jnp.where(is_lab, _NEG_INF, xs))
        acc_l = jnp.maximum(acc_l, jnp.where(is_lab, xs, _NEG_INF))
    accmax_ref[...] = acc_m
    acclab_ref[...] = acc_l

    @pl.when(i == TC_NCHUNK - 1)
    def _fin():
        outmax_ref[...] = jnp.max(acc_m, axis=1)
        outlab_ref[...] = jnp.max(acc_l, axis=1)


def _tc_call(logits, label):
    return pl.pallas_call(
        _tc_kernel,
        grid=(TC_NCHUNK,),
        in_specs=[
            pl.BlockSpec((B,), lambda i: (0,)),
            pl.BlockSpec((B, TC_CHUNK), lambda i: (0, i)),
        ],
        out_specs=[
            pl.BlockSpec((B,), lambda i: (0,)),
            pl.BlockSpec((B,), lambda i: (0,)),
        ],
        out_shape=[
            jax.ShapeDtypeStruct((B,), jnp.float32),
            jax.ShapeDtypeStruct((B,), jnp.float32),
        ],
        scratch_shapes=[
            pltpu.VMEM((B, LANES), jnp.float32),
            pltpu.VMEM((B, LANES), jnp.float32),
        ],
        compiler_params=pltpu.CompilerParams(
            dimension_semantics=("arbitrary",),
        ),
    )(label, logits)


def _sc_body(logits_hbm, labelb_hbm, outmax_hbm, outlab_hbm,
             labv, rowbuf, obuf_m, obuf_l):
    c = lax.axis_index("c")
    s = lax.axis_index("s")
    wid = s * SC_NC + c
    base = wid * ROWS_PER

    for j in range(ROWS_PER):
        row = base + j
        pltpu.sync_copy(logits_hbm.at[row, pl.ds(C0, W)], rowbuf)
        pltpu.sync_copy(labelb_hbm.at[row], labv)
        lab_vec = labv[...]
        cols0 = C0 + lax.iota(jnp.int32, SC_L)
        ninf = jnp.full((SC_L,), _NEG_INF, jnp.float32)

        def body(k, carry):
            acc_m, acc_l, cols = carry
            v = rowbuf[pl.ds(k * SC_L, SC_L)]
            is_lab = cols == lab_vec
            acc_m = jnp.maximum(acc_m, jnp.where(is_lab, ninf, v))
            acc_l = jnp.maximum(acc_l, jnp.where(is_lab, v, ninf))
            return acc_m, acc_l, cols + SC_L

        acc_m, acc_l, _ = lax.fori_loop(
            0, W // SC_L, body, (ninf, ninf, cols0))
        obuf_m[...] = acc_m
        obuf_l[...] = acc_l
        pltpu.sync_copy(obuf_m, outmax_hbm.at[row])
        pltpu.sync_copy(obuf_l, outlab_hbm.at[row])


_sc_call = functools.partial(
    pl.kernel,
    mesh=plsc.VectorSubcoreMesh(core_axis_name="c", subcore_axis_name="s"),
    out_type=[
        jax.ShapeDtypeStruct((B, SC_L), jnp.float32),
        jax.ShapeDtypeStruct((B, SC_L), jnp.float32),
    ],
    scratch_types=[
        pltpu.VMEM((SC_L,), jnp.int32),
        pltpu.VMEM((W,), jnp.float32),
        pltpu.VMEM((SC_L,), jnp.float32),
        pltpu.VMEM((SC_L,), jnp.float32),
    ],
)(_sc_body)


@jax.jit
def kernel(logits, label):
    tcmax, tclab = _tc_call(logits, label)
    label_bcast = jnp.broadcast_to(label[:, None], (B, SC_L))
    scmax, sclab = _sc_call(logits, label_bcast)
    max_other = jnp.maximum(tcmax, jnp.max(scmax, axis=1))
    logit_label = jnp.maximum(tclab, jnp.max(sclab, axis=1))
    diff = logit_label - max_other
    return -jnp.maximum(diff, 0.0)


# speedup vs baseline: 1.5937x; 1.5937x over previous
import functools

import jax
import jax.numpy as jnp
from jax import lax
from jax.experimental import pallas as pl
from jax.experimental.pallas import tpu as pltpu
from jax.experimental.pallas import tpu_sc as plsc

B = 128
V = 100000

TC_CHUNK = 4096
TC_NCHUNK = 12
C0 = TC_CHUNK * TC_NCHUNK
SC_END = (V // 128) * 128
TAIL = V - SC_END
TAIL_BLOCK = SC_END // 128

LANES = 128
FOLDS = TC_CHUNK // LANES
_NEG_INF = float("-inf")

SC_NC = 2
SC_NS = 16
NW = SC_NC * SC_NS
SC_L = 16

NTILES = (SC_END - C0) // 128
K = 10
NBATCH = 20
WTILES = K * NBATCH


def _tc_kernel(label_ref, logits_ref, tail_ref, outmax_ref, outlab_ref,
               accmax_ref, acclab_ref):
    i = pl.program_id(0)
    x = logits_ref[...]
    lane = jax.lax.broadcasted_iota(jnp.int32, (B, LANES), 1)
    lab = label_ref[...].reshape(B, 1)
    lab_rel = lab - i * TC_CHUNK

    @pl.when(i == 0)
    def _init():
        accmax_ref[...] = jnp.full((B, LANES), _NEG_INF, jnp.float32)
        acclab_ref[...] = jnp.full((B, LANES), _NEG_INF, jnp.float32)

    acc_m = accmax_ref[...]
    acc_l = acclab_ref[...]
    for k in range(FOLDS):
        xs = x[:, k * LANES:(k + 1) * LANES]
        is_lab = lane == (lab_rel - k * LANES)
        acc_m = jnp.maximum(acc_m, jnp.where(is_lab, _NEG_INF, xs))
        acc_l = jnp.maximum(acc_l, jnp.where(is_lab, xs, _NEG_INF))
    accmax_ref[...] = acc_m
    acclab_ref[...] = acc_l

    @pl.when(i == TC_NCHUNK - 1)
    def _fin():
        xt = tail_ref[...]
        is_lab_t = lane == (lab - SC_END)
        kill = jnp.logical_or(is_lab_t, lane >= TAIL)
        am = jnp.maximum(acc_m, jnp.where(kill, _NEG_INF, xt))
        al = jnp.maximum(acc_l, jnp.where(
            jnp.logical_and(is_lab_t, lane < TAIL), xt, _NEG_INF))
        outmax_ref[...] = jnp.max(am, axis=1)
        outlab_ref[...] = jnp.max(al, axis=1)


def _tc_call(logits, label):
    return pl.pallas_call(
        _tc_kernel,
        grid=(TC_NCHUNK,),
        in_specs=[
            pl.BlockSpec((B,), lambda i: (0,)),
            pl.BlockSpec((B, TC_CHUNK), lambda i: (0, i)),
            pl.BlockSpec((B, 128), lambda i: (0, TAIL_BLOCK)),
        ],
        out_specs=[
            pl.BlockSpec((B,), lambda i: (0,)),
            pl.BlockSpec((B,), lambda i: (0,)),
        ],
        out_shape=[
            jax.ShapeDtypeStruct((B,), jnp.float32),
            jax.ShapeDtypeStruct((B,), jnp.float32),
        ],
        scratch_shapes=[
            pltpu.VMEM((B, LANES), jnp.float32),
            pltpu.VMEM((B, LANES), jnp.float32),
        ],
        compiler_params=pltpu.CompilerParams(
            dimension_semantics=("arbitrary",),
        ),
    )(label, logits, logits)


def _sc_body(logits_hbm, labelf_hbm, outmax_hbm, outlab_hbm,
             labbuf, buf_a, buf_b, obuf_m, obuf_l, sem_a, sem_b):
    c = lax.axis_index("c")
    s = lax.axis_index("s")
    wid = s * SC_NC + c
    rb = wid // 2
    half = wid % 2
    tstart = half * (NTILES - WTILES)

    ninf = jnp.full((SC_L,), _NEG_INF, jnp.float32)
    lane = lax.iota(jnp.int32, SC_L)

    pltpu.sync_copy(labelf_hbm.at[pl.ds(rb * 8 * SC_L, 8 * SC_L)], labbuf)

    lab_tile = []
    lab_off = []
    lab_lane = []
    for r in range(8):
        lv = labbuf[pl.ds(r * SC_L, SC_L)]
        lab = lv[0]
        pos = lab - C0
        pos_c = jnp.where(pos >= 0, pos, jnp.int32(2 ** 30))
        t_glob = pos_c // 128
        t_loc = t_glob - tstart
        lab_tile.append(t_loc)
        lab_off.append((pos_c % 128) // SC_L * SC_L)
        lab_lane.append(pos_c % SC_L)

    for r in range(8):
        obuf_l[pl.ds(r * SC_L, SC_L)] = ninf

    bufs = (buf_a, buf_b)
    sems = (sem_a, sem_b)

    def issue(bidx, which):
        handles = []
        for ti in range(K):
            t = (tstart + bidx * K + ti) * 128 + C0
            handles.append(pltpu.async_copy(
                logits_hbm.at[pl.ds(rb * 8, 8), pl.ds(t, 128)],
                bufs[which].at[pl.ds(ti * 8, 8), :],
                sems[which]))
        return handles

    def process(bidx, which, accs):
        buf = bufs[which]
        base_t = bidx * K
        for ti in range(K):
            t = (tstart + bidx * K + ti) * 128 + C0
            pltpu.make_async_copy(
                logits_hbm.at[pl.ds(rb * 8, 8), pl.ds(t, 128)],
                buf.at[pl.ds(ti * 8, 8), :],
                sems[which]).wait()
        for r in range(8):
            tl = lab_tile[r]
            inb = jnp.logical_and(tl >= base_t, tl < base_t + K)

            @pl.when(inb)
            def _zap(r=r, tl=tl):
                row = (tl - base_t) * 8 + r
                v = buf[row, pl.ds(lab_off[r], SC_L)]
                hit = lane == lab_lane[r]
                obuf_l[pl.ds(r * SC_L, SC_L)] = jnp.where(hit, v, ninf)
                buf[row, pl.ds(lab_off[r], SC_L)] = jnp.where(hit, ninf, v)

        def tbody(t, accs):
            new = []
            for r in range(8):
                a = accs[r]
                for h in range(8):
                    a = jnp.maximum(a, buf[t * 8 + r, pl.ds(h * SC_L, SC_L)])
                new.append(a)
            return tuple(new)

        return lax.fori_loop(0, K, tbody, accs)

    accs = (ninf,) * 8
    issue(0, 0)

    def pair(p, accs):
        issue(2 * p + 1, 1)
        accs = process(2 * p, 0, accs)

        @pl.when(2 * p + 2 < NBATCH)
        def _next():
            issue(2 * p + 2, 0)

        accs = process(2 * p + 1, 1, accs)
        return accs

    accs = lax.fori_loop(0, NBATCH // 2, pair, accs)

    for r in range(8):
        obuf_m[pl.ds(r * SC_L, SC_L)] = accs[r]

    out_off = (half * B + rb * 8) * SC_L
    pltpu.sync_copy(obuf_m, outmax_hbm.at[pl.ds(out_off, 8 * SC_L)])
    pltpu.sync_copy(obuf_l, outlab_hbm.at[pl.ds(out_off, 8 * SC_L)])


_sc_call = functools.partial(
    pl.kernel,
    mesh=plsc.VectorSubcoreMesh(core_axis_name="c", subcore_axis_name="s"),
    out_type=[
        jax.ShapeDtypeStruct((2 * B * SC_L,), jnp.float32),
        jax.ShapeDtypeStruct((2 * B * SC_L,), jnp.float32),
    ],
    scratch_types=[
        pltpu.VMEM((8 * SC_L,), jnp.int32),
        pltpu.VMEM((K * 8, 128), jnp.float32),
        pltpu.VMEM((K * 8, 128), jnp.float32),
        pltpu.VMEM((8 * SC_L,), jnp.float32),
        pltpu.VMEM((8 * SC_L,), jnp.float32),
        pltpu.SemaphoreType.DMA,
        pltpu.SemaphoreType.DMA,
    ],
)(_sc_body)


@jax.jit
def kernel(logits, label):
    label_bcast = jnp.broadcast_to(label[:, None], (B, SC_L)).reshape(B * SC_L)
    scmax, sclab = _sc_call(logits, label_bcast)
    tcmax, tclab = _tc_call(logits, label)
    scmax = scmax.reshape(2, B, SC_L)
    sclab = sclab.reshape(2, B, SC_L)
    max_other = jnp.maximum(tcmax, jnp.max(scmax, axis=(0, 2)))
    logit_label = jnp.maximum(tclab, jnp.max(sclab, axis=(0, 2)))
    diff = logit_label - max_other
    return -jnp.maximum(diff, 0.0)


# speedup vs baseline: 3.3218x; 2.0844x over previous
import functools

import jax
import jax.numpy as jnp
from jax import lax
from jax.experimental import pallas as pl
from jax.experimental.pallas import tpu as pltpu
from jax.experimental.pallas import tpu_sc as plsc

B = 128
V = 100000
_NEG_INF = float("-inf")

SC_NC = 2
SC_NS = 16
NW = SC_NC * SC_NS
SC_L = 16

T = 80
NB = 16
WROWS = T * NB
SC_ROWS = NW * WROWS
C0 = V - SC_ROWS

CHUNK = 5904
NCHUNK = C0 // CHUNK


def _tc_kernel(label_ref, logits_ref, out_ref, acc_ref):
    i = pl.program_id(0)
    x = logits_ref[...]
    lab = label_ref[...].reshape(1, B)
    riota = jax.lax.broadcasted_iota(jnp.int32, (CHUNK, B), 0)
    is_lab = riota == (lab - i * CHUNK)
    m = jnp.max(jnp.where(is_lab, _NEG_INF, x), axis=0).reshape(1, B)

    @pl.when(i == 0)
    def _init():
        acc_ref[...] = m

    @pl.when(i > 0)
    def _acc():
        acc_ref[...] = jnp.maximum(acc_ref[...], m)

    @pl.when(i == NCHUNK - 1)
    def _fin():
        out_ref[...] = acc_ref[...].reshape(B)


def _tc_call(logits_t, label):
    return pl.pallas_call(
        _tc_kernel,
        grid=(NCHUNK,),
        in_specs=[
            pl.BlockSpec((B,), lambda i: (0,)),
            pl.BlockSpec((CHUNK, B), lambda i: (i, 0)),
        ],
        out_specs=pl.BlockSpec((B,), lambda i: (0,)),
        out_shape=jax.ShapeDtypeStruct((B,), jnp.float32),
        scratch_shapes=[pltpu.VMEM((1, B), jnp.float32)],
        compiler_params=pltpu.CompilerParams(
            dimension_semantics=("arbitrary",),
        ),
    )(label, logits_t)


def _sc_body(logits_hbm, labelf_hbm, outm1_hbm, outm2_hbm, outg_hbm,
             labbuf, buf_a, buf_b, obuf, gbuf, sem_a, sem_b):
    c = lax.axis_index("c")
    s = lax.axis_index("s")
    wid = s * SC_NC + c
    row0 = C0 + wid * WROWS

    ninf = jnp.full((SC_L,), _NEG_INF, jnp.float32)
    lane = lax.iota(jnp.int32, SC_L)

    pltpu.sync_copy(labelf_hbm.at[pl.ds(wid * 4 * SC_L, 4 * SC_L)], labbuf)
    for j in range(4):
        lv = labbuf[pl.ds(j * SC_L, SC_L)]
        lab = lv[0]
        rlane = wid * 4 + j
        gbase = (rlane // SC_L) * SC_L
        pltpu.sync_copy(
            logits_hbm.at[pl.ds(lab, 1), :],
            gbuf.at[pl.ds(j, 1), :])
        v = gbuf[j, pl.ds(gbase, SC_L)]
        hit = lane == (rlane % SC_L)
        obuf[pl.ds((16 + j) * SC_L, SC_L)] = jnp.where(hit, v, ninf)

    bufs = (buf_a, buf_b)
    sems = (sem_a, sem_b)

    def issue(bidx, which):
        pltpu.async_copy(
            logits_hbm.at[pl.ds(row0 + bidx * T, T), :],
            bufs[which], sems[which])

    def process(bidx, which, accs):
        buf = bufs[which]
        pltpu.make_async_copy(
            logits_hbm.at[pl.ds(row0 + bidx * T, T), :],
            buf, sems[which]).wait()

        def tbody(t, accs):
            new = []
            for h in range(8):
                m1, m2 = accs[2 * h], accs[2 * h + 1]
                x = buf[t, pl.ds(h * SC_L, SC_L)]
                m2 = jnp.maximum(m2, jnp.minimum(m1, x))
                m1 = jnp.maximum(m1, x)
                new.append(m1)
                new.append(m2)
            return tuple(new)

        return lax.fori_loop(0, T, tbody, accs)

    accs = (ninf,) * 16
    issue(0, 0)

    def pair(p, accs):
        issue(2 * p + 1, 1)
        accs = process(2 * p, 0, accs)

        @pl.when(2 * p + 2 < NB)
        def _next():
            issue(2 * p + 2, 0)

        accs = process(2 * p + 1, 1, accs)
        return accs

    accs = lax.fori_loop(0, NB // 2, pair, accs)

    for h in range(8):
        obuf[pl.ds(h * SC_L, SC_L)] = accs[2 * h]
        obuf[pl.ds((8 + h) * SC_L, SC_L)] = accs[2 * h + 1]

    pltpu.sync_copy(obuf.at[pl.ds(0, B)], outm1_hbm.at[pl.ds(wid * B, B)])
    pltpu.sync_copy(obuf.at[pl.ds(B, B)], outm2_hbm.at[pl.ds(wid * B, B)])
    pltpu.sync_copy(obuf.at[pl.ds(2 * B, 4 * SC_L)],
                    outg_hbm.at[pl.ds(wid * 4 * SC_L, 4 * SC_L)])


_sc_call = functools.partial(
    pl.kernel,
    mesh=plsc.VectorSubcoreMesh(core_axis_name="c", subcore_axis_name="s"),
    out_type=[
        jax.ShapeDtypeStruct((NW * B,), jnp.float32),
        jax.ShapeDtypeStruct((NW * B,), jnp.float32),
        jax.ShapeDtypeStruct((NW * 4 * SC_L,), jnp.float32),
    ],
    scratch_types=[
        pltpu.VMEM((4 * SC_L,), jnp.int32),
        pltpu.VMEM((T, B), jnp.float32),
        pltpu.VMEM((T, B), jnp.float32),
        pltpu.VMEM((2 * B + 4 * SC_L,), jnp.float32),
        pltpu.VMEM((4, B), jnp.float32),
        pltpu.SemaphoreType.DMA,
        pltpu.SemaphoreType.DMA,
    ],
)(_sc_body)


@jax.jit
def kernel(logits, label):
    logits_t = logits.T
    label_bcast = jnp.broadcast_to(label[:, None], (B, SC_L)).reshape(B * SC_L)
    scm1, scm2, scg = _sc_call(logits_t, label_bcast)
    tcmax = _tc_call(logits_t, label)

    loglab = jnp.max(scg.reshape(B, SC_L), axis=1)
    m1s = scm1.reshape(NW, B)
    m2s = scm2.reshape(NW, B)
    m1 = jnp.full((B,), _NEG_INF, jnp.float32)
    m2 = jnp.full((B,), _NEG_INF, jnp.float32)
    for w in range(NW):
        x = m1s[w]
        m2 = jnp.maximum(m2, jnp.minimum(m1, x))
        m1 = jnp.maximum(m1, x)
    m2 = jnp.maximum(m2, jnp.max(m2s, axis=0))
    m_sc = jnp.where(jnp.logical_or(label < C0, loglab < m1), m1, m2)
    max_other = jnp.maximum(tcmax, m_sc)
    return -jnp.maximum(loglab - max_other, 0.0)


# speedup vs baseline: 3.4741x; 1.0458x over previous
import functools

import jax
import jax.numpy as jnp
from jax import lax
from jax.experimental import pallas as pl
from jax.experimental.pallas import tpu as pltpu
from jax.experimental.pallas import tpu_sc as plsc

B = 128
V = 100000
_NEG_INF = float("-inf")

SC_NC = 2
SC_NS = 16
NW = SC_NC * SC_NS
SC_L = 16

T = 80
NB = 8
WROWS = T * NB
SC_ROWS = NW * WROWS
C0 = V - SC_ROWS

CHUNK = 7952
NCHUNK = C0 // CHUNK


def _tc_kernel(label_ref, logits_ref, out_ref, acc_ref):
    i = pl.program_id(0)
    x = logits_ref[...]
    lab = label_ref[...].reshape(1, B)
    riota = jax.lax.broadcasted_iota(jnp.int32, (CHUNK, B), 0)
    is_lab = riota == (lab - i * CHUNK)
    m = jnp.max(jnp.where(is_lab, _NEG_INF, x), axis=0).reshape(1, B)

    @pl.when(i == 0)
    def _init():
        acc_ref[...] = m

    @pl.when(i > 0)
    def _acc():
        acc_ref[...] = jnp.maximum(acc_ref[...], m)

    @pl.when(i == NCHUNK - 1)
    def _fin():
        out_ref[...] = acc_ref[...].reshape(B)


def _tc_call(logits_t, label):
    return pl.pallas_call(
        _tc_kernel,
        grid=(NCHUNK,),
        in_specs=[
            pl.BlockSpec((B,), lambda i: (0,)),
            pl.BlockSpec((CHUNK, B), lambda i: (i, 0)),
        ],
        out_specs=pl.BlockSpec((B,), lambda i: (0,)),
        out_shape=jax.ShapeDtypeStruct((B,), jnp.float32),
        scratch_shapes=[pltpu.VMEM((1, B), jnp.float32)],
        compiler_params=pltpu.CompilerParams(
            dimension_semantics=("arbitrary",),
        ),
    )(label, logits_t)


def _sc_body(logits_hbm, labelf_hbm, outm1_hbm, outm2_hbm, outg_hbm,
             labbuf, buf_a, buf_b, obuf, gbuf, sem_a, sem_b):
    c = lax.axis_index("c")
    s = lax.axis_index("s")
    wid = s * SC_NC + c
    row0 = C0 + wid * WROWS

    ninf = jnp.full((SC_L,), _NEG_INF, jnp.float32)
    lane = lax.iota(jnp.int32, SC_L)

    pltpu.sync_copy(labelf_hbm.at[pl.ds(wid * 4 * SC_L, 4 * SC_L)], labbuf)
    for j in range(4):
        lv = labbuf[pl.ds(j * SC_L, SC_L)]
        lab = lv[0]
        rlane = wid * 4 + j
        gbase = (rlane // SC_L) * SC_L
        pltpu.sync_copy(
            logits_hbm.at[pl.ds(lab, 1), :],
            gbuf.at[pl.ds(j, 1), :])
        v = gbuf[j, pl.ds(gbase, SC_L)]
        hit = lane == (rlane % SC_L)
        obuf[pl.ds((16 + j) * SC_L, SC_L)] = jnp.where(hit, v, ninf)

    bufs = (buf_a, buf_b)
    sems = (sem_a, sem_b)

    def issue(bidx, which):
        pltpu.async_copy(
            logits_hbm.at[pl.ds(row0 + bidx * T, T), :],
            bufs[which], sems[which])

    def process(bidx, which, accs):
        buf = bufs[which]
        pltpu.make_async_copy(
            logits_hbm.at[pl.ds(row0 + bidx * T, T), :],
            buf, sems[which]).wait()

        def tbody(t, accs):
            new = []
            for h in range(8):
                m1, m2 = accs[2 * h], accs[2 * h + 1]
                x = buf[t, pl.ds(h * SC_L, SC_L)]
                m2 = jnp.maximum(m2, jnp.minimum(m1, x))
                m1 = jnp.maximum(m1, x)
                new.append(m1)
                new.append(m2)
            return tuple(new)

        return lax.fori_loop(0, T, tbody, accs)

    accs = (ninf,) * 16
    issue(0, 0)

    def pair(p, accs):
        issue(2 * p + 1, 1)
        accs = process(2 * p, 0, accs)

        @pl.when(2 * p + 2 < NB)
        def _next():
            issue(2 * p + 2, 0)

        accs = process(2 * p + 1, 1, accs)
        return accs

    accs = lax.fori_loop(0, NB // 2, pair, accs)

    for h in range(8):
        obuf[pl.ds(h * SC_L, SC_L)] = accs[2 * h]
        obuf[pl.ds((8 + h) * SC_L, SC_L)] = accs[2 * h + 1]

    pltpu.sync_copy(obuf.at[pl.ds(0, B)], outm1_hbm.at[pl.ds(wid * B, B)])
    pltpu.sync_copy(obuf.at[pl.ds(B, B)], outm2_hbm.at[pl.ds(wid * B, B)])
    pltpu.sync_copy(obuf.at[pl.ds(2 * B, 4 * SC_L)],
                    outg_hbm.at[pl.ds(wid * 4 * SC_L, 4 * SC_L)])


_sc_call = functools.partial(
    pl.kernel,
    mesh=plsc.VectorSubcoreMesh(core_axis_name="c", subcore_axis_name="s"),
    out_type=[
        jax.ShapeDtypeStruct((NW * B,), jnp.float32),
        jax.ShapeDtypeStruct((NW * B,), jnp.float32),
        jax.ShapeDtypeStruct((NW * 4 * SC_L,), jnp.float32),
    ],
    scratch_types=[
        pltpu.VMEM((4 * SC_L,), jnp.int32),
        pltpu.VMEM((T, B), jnp.float32),
        pltpu.VMEM((T, B), jnp.float32),
        pltpu.VMEM((2 * B + 4 * SC_L,), jnp.float32),
        pltpu.VMEM((4, B), jnp.float32),
        pltpu.SemaphoreType.DMA,
        pltpu.SemaphoreType.DMA,
    ],
)(_sc_body)


@jax.jit
def kernel(logits, label):
    logits_t = logits.T
    label_bcast = jnp.broadcast_to(label[:, None], (B, SC_L)).reshape(B * SC_L)
    scm1, scm2, scg = _sc_call(logits_t, label_bcast)
    tcmax = _tc_call(logits_t, label)

    loglab = jnp.max(scg.reshape(B, SC_L), axis=1)
    m1s = scm1.reshape(NW, B)
    m2s = scm2.reshape(NW, B)
    m1 = jnp.full((B,), _NEG_INF, jnp.float32)
    m2 = jnp.full((B,), _NEG_INF, jnp.float32)
    for w in range(NW):
        x = m1s[w]
        m2 = jnp.maximum(m2, jnp.minimum(m1, x))
        m1 = jnp.maximum(m1, x)
    m2 = jnp.maximum(m2, jnp.max(m2s, axis=0))
    m_sc = jnp.where(jnp.logical_or(label < C0, loglab < m1), m1, m2)
    max_other = jnp.maximum(tcmax, m_sc)
    return -jnp.maximum(loglab - max_other, 0.0)
